# baseline (speedup 1.0000x reference)
import functools

import jax
import jax.numpy as jnp
from jax import lax
from jax.experimental import pallas as pl
from jax.experimental.pallas import tpu as pltpu
from jax.experimental.pallas import tpu_sc as plsc

N = 10000
E = 160000
IN = 256
H = 512
LAT = 128
MAXN = 32
B = 64

NC = 2
NS = 16
KW = 32
NR = 10240
STRIPE = NR // NS
EPAD = 163840
FC = 128
NFC = H // FC
BN = 400
NB = N // BN

_mesh = plsc.VectorSubcoreMesh(core_axis_name="c", subcore_axis_name="s")


DKW = 128
DWIN = EPAD // (NC * NS) // DKW


def _deg_body(dst128_hbm, ones_hbm, zeros_hbm, dega, degb,
              acc, ones_v, idx_v, sem_s):
    c = lax.axis_index("c")
    s = lax.axis_index("s")
    t = c * NS + s
    pltpu.sync_copy(ones_hbm, ones_v)
    pltpu.sync_copy(zeros_hbm.at[pl.ds(s * STRIPE, STRIPE)],
                    acc.at[pl.ds(s * STRIPE, STRIPE)])
    pltpu.sync_copy(dst128_hbm.at[pl.ds(t * DWIN, DWIN)], idx_v)
    plsc.subcore_barrier()

    def fire(i, carry):
        pltpu.async_copy(ones_v, acc.at[idx_v.at[i]], sem_s, add=True)
        return carry

    lax.fori_loop(0, DWIN, fire, 0)

    def drain(i, carry):
        pltpu.make_async_copy(zeros_hbm.at[pl.ds(0, DKW)], ones_v,
                              sem_s).wait()
        return carry

    lax.fori_loop(0, DWIN, drain, 0)
    plsc.subcore_barrier()

    @pl.when(c == 0)
    def _():
        pltpu.sync_copy(acc.at[pl.ds(s * STRIPE, STRIPE)],
                        dega.at[pl.ds(s * STRIPE, STRIPE)])

    @pl.when(c == 1)
    def _():
        pltpu.sync_copy(acc.at[pl.ds(s * STRIPE, STRIPE)],
                        degb.at[pl.ds(s * STRIPE, STRIPE)])


_deg_call = pl.kernel(
    _deg_body,
    out_type=[jax.ShapeDtypeStruct((NR,), jnp.float32),
              jax.ShapeDtypeStruct((NR,), jnp.float32)],
    mesh=_mesh,
    scratch_types=[
        pltpu.VMEM_SHARED((NR,), jnp.float32),
        pltpu.VMEM((DKW,), jnp.float32),
        pltpu.VMEM((DWIN, DKW), jnp.int32),
        pltpu.SemaphoreType.DMA,
    ],
)


NWIN = EPAD // NS // KW
IB = 32
NBUF = 8
GD = 4
SD = 4


def _agg_chunk(p_hbm, src2_hbm, dst2_hbm, out_hbm,
               acc, rows_v, sidx_v, didx_v, sem_g, sem_s, s):
    @pl.when(s < NS - 1)
    def _():
        pltpu.sync_copy(p_hbm.at[pl.ds(s * STRIPE, STRIPE)],
                        acc.at[pl.ds(s * STRIPE, STRIPE)])

    @pl.when(s == NS - 1)
    def _():
        pltpu.sync_copy(p_hbm.at[pl.ds((NS - 1) * STRIPE, N - (NS - 1) * STRIPE)],
                        acc.at[pl.ds((NS - 1) * STRIPE, N - (NS - 1) * STRIPE)])

    def load_idx(bi, buf):
        pltpu.sync_copy(src2_hbm.at[pl.ds(s * NWIN + bi * IB, IB)],
                        sidx_v.at[buf])
        pltpu.sync_copy(dst2_hbm.at[pl.ds(s * NWIN + bi * IB, IB)],
                        didx_v.at[buf])

    def start_gather(w, buf):
        pltpu.async_copy(p_hbm.at[sidx_v.at[(w // IB) % 2, w % IB]],
                         rows_v.at[buf], sem_g)

    def wait_gather(buf):
        pltpu.make_async_copy(p_hbm.at[pl.ds(0, KW)], rows_v.at[buf],
                              sem_g).wait()

    def start_scatter(w, buf):
        pltpu.async_copy(rows_v.at[buf],
                         acc.at[didx_v.at[(w // IB) % 2, w % IB]], sem_s,
                         add=True)

    def wait_scatter(buf):
        pltpu.make_async_copy(p_hbm.at[pl.ds(0, KW)], rows_v.at[buf],
                              sem_s).wait()

    load_idx(0, 0)
    plsc.subcore_barrier()
    for k in range(GD):
        start_gather(k, k)

    def body(g, carry):
        for j in range(NBUF):
            w = g * NBUF + j
            wait_gather(j)

            @pl.when(w >= SD)
            def _():
                wait_scatter((j + NBUF - SD) % NBUF)

            @pl.when((w + GD < NWIN) & ((w + GD) % IB == 0))
            def _():
                load_idx((w + GD) // IB, ((w + GD) // IB) % 2)

            @pl.when(w + GD < NWIN)
            def _():
                start_gather(w + GD, (j + GD) % NBUF)

            start_scatter(w, j)
        return carry

    lax.fori_loop(0, NWIN // NBUF, body, 0)
    for k in range(SD):
        wait_scatter(NBUF - SD + k)
    plsc.subcore_barrier()
    pltpu.sync_copy(acc.at[pl.ds(s * STRIPE, STRIPE)],
                    out_hbm.at[pl.ds(s * STRIPE, STRIPE)])
    plsc.subcore_barrier()


def _agg_body(p0, p1, p2, p3, src2_hbm, dst2_hbm,
              a0, a1, a2, a3,
              acc, rows_v, sidx_v, didx_v, sem_g, sem_s):
    c = lax.axis_index("c")
    s = lax.axis_index("s")

    @pl.when(c == 0)
    def _():
        _agg_chunk(p0, src2_hbm, dst2_hbm, a0, acc, rows_v,
                   sidx_v, didx_v, sem_g, sem_s, s)
        _agg_chunk(p1, src2_hbm, dst2_hbm, a1, acc, rows_v,
                   sidx_v, didx_v, sem_g, sem_s, s)

    @pl.when(c == 1)
    def _():
        _agg_chunk(p2, src2_hbm, dst2_hbm, a2, acc, rows_v,
                   sidx_v, didx_v, sem_g, sem_s, s)
        _agg_chunk(p3, src2_hbm, dst2_hbm, a3, acc, rows_v,
                   sidx_v, didx_v, sem_g, sem_s, s)


_agg_call = pl.kernel(
    _agg_body,
    out_type=[jax.ShapeDtypeStruct((NR, FC), jnp.float32)] * NFC,
    mesh=_mesh,
    scratch_types=[
        pltpu.VMEM_SHARED((NR, FC), jnp.float32),
        pltpu.VMEM((NBUF, KW, FC), jnp.float32),
        pltpu.VMEM((2, IB, KW), jnp.int32),
        pltpu.VMEM((2, IB, KW), jnp.int32),
        pltpu.SemaphoreType.DMA,
        pltpu.SemaphoreType.DMA,
    ],
)


def _mm1_body(x_ref, w_ref, dega_ref, degb_ref, o0, o1, o2, o3):
    dis = lax.rsqrt(dega_ref[0, 0, :] + degb_ref[0, 0, :] + 1.0)
    h = jnp.dot(x_ref[...], w_ref[...], preferred_element_type=jnp.float32)
    p = h * dis[:, None]
    o0[...] = p[:, 0 * FC:1 * FC]
    o1[...] = p[:, 1 * FC:2 * FC]
    o2[...] = p[:, 2 * FC:3 * FC]
    o3[...] = p[:, 3 * FC:4 * FC]


def _mm1(x, W1, dega3, degb3):
    return pl.pallas_call(
        _mm1_body,
        grid=(NB,),
        in_specs=[
            pl.BlockSpec((BN, IN), lambda i: (i, 0)),
            pl.BlockSpec((IN, H), lambda i: (0, 0)),
            pl.BlockSpec((1, 1, BN), lambda i: (i, 0, 0)),
            pl.BlockSpec((1, 1, BN), lambda i: (i, 0, 0)),
        ],
        out_specs=[pl.BlockSpec((BN, FC), lambda i: (i, 0))] * NFC,
        out_shape=[jax.ShapeDtypeStruct((N, FC), jnp.float32)] * NFC,
    )(x, W1, dega3, degb3)


def _mm2_body(a0, a1, a2, a3, dega_ref, degb_ref,
              b_ref, w_ref, o0, o1, o2, o3):
    dis = lax.rsqrt(dega_ref[0, 0, :] + degb_ref[0, 0, :] + 1.0)
    hcat = jnp.concatenate([a0[...], a1[...], a2[...], a3[...]], axis=1)
    h1 = jnp.maximum(hcat * dis[:, None] + b_ref[0, :], 0.0)
    h2 = jnp.dot(h1, w_ref[...], preferred_element_type=jnp.float32)
    p = h2 * dis[:, None]
    o0[...] = p[:, 0 * FC:1 * FC]
    o1[...] = p[:, 1 * FC:2 * FC]
    o2[...] = p[:, 2 * FC:3 * FC]
    o3[...] = p[:, 3 * FC:4 * FC]


def _mm2(aggs, dega3, degb3, b1r, W2):
    return pl.pallas_call(
        _mm2_body,
        grid=(NB,),
        in_specs=(
            [pl.BlockSpec((BN, FC), lambda i: (i, 0))] * NFC
            + [pl.BlockSpec((1, 1, BN), lambda i: (i, 0, 0))] * 2
            + [pl.BlockSpec((1, H), lambda i: (0, 0)),
               pl.BlockSpec((H, H), lambda i: (0, 0))]
        ),
        out_specs=[pl.BlockSpec((BN, FC), lambda i: (i, 0))] * NFC,
        out_shape=[jax.ShapeDtypeStruct((N, FC), jnp.float32)] * NFC,
    )(*aggs, dega3, degb3, b1r, W2)


def _pool_body(a0, a1, a2, a3, dega_ref, degb_ref,
               b_ref, batch_ref, wmu_ref, bmu_ref, wlv_ref, blv_ref,
               wd1_ref, bd1_ref, eps_ref,
               hd_out, mu_out, lv_out, s_scr, c_scr):
    i = pl.program_id(0)

    @pl.when(i == 0)
    def _():
        s_scr[...] = jnp.zeros_like(s_scr)
        c_scr[...] = jnp.zeros_like(c_scr)

    dis = lax.rsqrt(dega_ref[0, 0, :] + degb_ref[0, 0, :] + 1.0)
    hcat = jnp.concatenate([a0[...], a1[...], a2[...], a3[...]], axis=1)
    h2 = jnp.maximum(hcat * dis[:, None] + b_ref[0, :], 0.0)
    bt = batch_ref[0, 0, :]
    oh = (bt[None, :] == lax.broadcasted_iota(jnp.int32, (B, BN), 0)
          ).astype(jnp.float32)
    s_scr[...] += jnp.dot(oh, h2, preferred_element_type=jnp.float32)
    c_scr[...] += jnp.sum(oh, axis=1, keepdims=True)

    @pl.when(i == NB - 1)
    def _():
        g = s_scr[...] / jnp.maximum(c_scr[:, 0:1], 1.0)
        mu = jnp.dot(g, wmu_ref[...], preferred_element_type=jnp.float32) + bmu_ref[0, :]
        lv = jnp.dot(g, wlv_ref[...], preferred_element_type=jnp.float32) + blv_ref[0, :]
        std = jnp.exp(0.5 * lv)
        z = mu + eps_ref[...] * std
        hd = jnp.maximum(
            jnp.dot(z, wd1_ref[...], preferred_element_type=jnp.float32) + bd1_ref[0, :],
            0.0)
        hd_out[...] = hd
        mu_out[...] = mu
        lv_out[...] = lv


def _pool(aggs, dega3, degb3, b2r, batch3, Wmu, bmur, Wlv, blvr,
          Wd1, bd1r, eps):
    full = lambda a, b: pl.BlockSpec((a, b), lambda i: (0, 0))
    return pl.pallas_call(
        _pool_body,
        grid=(NB,),
        in_specs=(
            [pl.BlockSpec((BN, FC), lambda i: (i, 0))] * NFC
            + [pl.BlockSpec((1, 1, BN), lambda i: (i, 0, 0))] * 2
            + [full(1, H), pl.BlockSpec((1, 1, BN), lambda i: (i, 0, 0)),
               full(H, LAT), full(1, LAT), full(H, LAT), full(1, LAT),
               full(LAT, H), full(1, H), full(B, LAT)]
        ),
        out_specs=[full(B, H), full(B, LAT), full(B, LAT)],
        out_shape=[jax.ShapeDtypeStruct((B, H), jnp.float32),
                   jax.ShapeDtypeStruct((B, LAT), jnp.float32),
                   jax.ShapeDtypeStruct((B, LAT), jnp.float32)],
        scratch_shapes=[pltpu.VMEM((B, H), jnp.float32),
                        pltpu.VMEM((B, FC), jnp.float32)],
    )(*aggs, dega3, degb3, b2r, batch3, Wmu, bmur, Wlv, blvr,
      Wd1, bd1r, eps)


_NXC = 8
_DW = MAXN * IN // _NXC


def _dec_body(hd_ref, wn_ref, bn_ref, wa_ref, ba_ref, xr_out, adj_out):
    j = pl.program_id(0)

    @pl.when(j < _NXC)
    def _():
        xr_out[...] = jnp.dot(hd_ref[...], wn_ref[...],
                              preferred_element_type=jnp.float32) + bn_ref[0, :]

    @pl.when(j == _NXC)
    def _():
        a = jnp.dot(hd_ref[...], wa_ref[...],
                    preferred_element_type=jnp.float32) + ba_ref[0, :]
        sg = 1.0 / (1.0 + jnp.exp(-a))
        ci = lax.broadcasted_iota(jnp.int32, (B, MAXN * MAXN), 1)
        diag = (ci // MAXN) == (ci % MAXN)
        adj_out[...] = jnp.where(diag, 0.0, sg)


def _dec(hd, Wn, bnr, Wa, bar):
    cap = lambda j: (0, jnp.minimum(j, _NXC - 1))
    return pl.pallas_call(
        _dec_body,
        grid=(_NXC + 1,),
        in_specs=[
            pl.BlockSpec((B, H), lambda j: (0, 0)),
            pl.BlockSpec((H, _DW), lambda j: cap(j)),
            pl.BlockSpec((1, _DW), lambda j: cap(j)),
            pl.BlockSpec((H, MAXN * MAXN), lambda j: (0, 0)),
            pl.BlockSpec((1, MAXN * MAXN), lambda j: (0, 0)),
        ],
        out_specs=[pl.BlockSpec((B, _DW), lambda j: cap(j)),
                   pl.BlockSpec((B, MAXN * MAXN), lambda j: (0, 0))],
        out_shape=[jax.ShapeDtypeStruct((B, MAXN * IN), jnp.float32),
                   jax.ShapeDtypeStruct((B, MAXN * MAXN), jnp.float32)],
    )(hd, Wn, bnr, Wa, bar)


def kernel(x, edge_index, batch, W1, b1, W2, b2, Wmu, bmu, Wlv, blv,
           Wd1, bd1, Wn, bn, Wa, ba):
    npad = EPAD - E
    fill = jnp.arange(npad, dtype=jnp.int32)
    src_pad = jnp.concatenate([edge_index[0], fill % N])
    dst_pad = jnp.concatenate([edge_index[1], N + fill % (NR - N)])

    zeros_1d = jnp.zeros((NR,), jnp.float32)
    ones_w = jnp.ones((DKW,), jnp.float32)

    dega, degb = _deg_call(dst_pad.reshape(-1, DKW), ones_w, zeros_1d)
    dega3 = dega[:N].reshape(NB, 1, BN)
    degb3 = degb[:N].reshape(NB, 1, BN)

    src2 = src_pad.reshape(-1, KW)
    dst2 = dst_pad.reshape(-1, KW)
    ps1 = _mm1(x, W1, dega3, degb3)
    aggs1 = _agg_call(*ps1, src2, dst2)

    ps2 = _mm2(aggs1, dega3, degb3, b1.reshape(1, H), W2)
    aggs2 = _agg_call(*ps2, src2, dst2)

    eps = jax.random.normal(jax.random.key(42), (B, LAT), jnp.float32)
    batch3 = batch.reshape(NB, 1, BN)
    hd, mu, logvar = _pool(aggs2, dega3, degb3, b2.reshape(1, H),
                           batch3, Wmu, bmu.reshape(1, LAT), Wlv,
                           blv.reshape(1, LAT), Wd1, bd1.reshape(1, H), eps)

    xr, adj = _dec(hd, Wn, bn.reshape(1, MAXN * IN), Wa,
                   ba.reshape(1, MAXN * MAXN))
    return (xr.reshape(B, MAXN, IN), adj.reshape(B, MAXN, MAXN), mu, logvar)


# speedup vs baseline: 1.0180x; 1.0180x over previous
import functools

import jax
import jax.numpy as jnp
from jax import lax
from jax.experimental import pallas as pl
from jax.experimental.pallas import tpu as pltpu
from jax.experimental.pallas import tpu_sc as plsc

N = 10000
E = 160000
IN = 256
H = 512
LAT = 128
MAXN = 32
B = 64

NC = 2
NS = 16
KW = 40
NR = 10240
STRIPE = NR // NS
EPAD = 163840
FC = 128
NFC = H // FC
BN = 400
NB = N // BN

_mesh = plsc.VectorSubcoreMesh(core_axis_name="c", subcore_axis_name="s")


DKW = 128
DWIN = EPAD // (NC * NS) // DKW


def _deg_body(dst128_hbm, ones_hbm, zeros_hbm, dega, degb,
              acc, ones_v, idx_v, sem_s):
    c = lax.axis_index("c")
    s = lax.axis_index("s")
    t = c * NS + s
    pltpu.sync_copy(ones_hbm, ones_v)
    pltpu.sync_copy(zeros_hbm.at[pl.ds(s * STRIPE, STRIPE)],
                    acc.at[pl.ds(s * STRIPE, STRIPE)])
    pltpu.sync_copy(dst128_hbm.at[pl.ds(t * DWIN, DWIN)], idx_v)
    plsc.subcore_barrier()

    def fire(i, carry):
        pltpu.async_copy(ones_v, acc.at[idx_v.at[i]], sem_s, add=True)
        return carry

    lax.fori_loop(0, DWIN, fire, 0)

    def drain(i, carry):
        pltpu.make_async_copy(zeros_hbm.at[pl.ds(0, DKW)], ones_v,
                              sem_s).wait()
        return carry

    lax.fori_loop(0, DWIN, drain, 0)
    plsc.subcore_barrier()

    @pl.when(c == 0)
    def _():
        pltpu.sync_copy(acc.at[pl.ds(s * STRIPE, STRIPE)],
                        dega.at[pl.ds(s * STRIPE, STRIPE)])

    @pl.when(c == 1)
    def _():
        pltpu.sync_copy(acc.at[pl.ds(s * STRIPE, STRIPE)],
                        degb.at[pl.ds(s * STRIPE, STRIPE)])


_deg_call = pl.kernel(
    _deg_body,
    out_type=[jax.ShapeDtypeStruct((NR,), jnp.float32),
              jax.ShapeDtypeStruct((NR,), jnp.float32)],
    mesh=_mesh,
    scratch_types=[
        pltpu.VMEM_SHARED((NR,), jnp.float32),
        pltpu.VMEM((DKW,), jnp.float32),
        pltpu.VMEM((DWIN, DKW), jnp.int32),
        pltpu.SemaphoreType.DMA,
    ],
)


NWIN = EPAD // NS // KW
IB = 16
NBUF = 8
GD = 5
SD = 3


def _agg_chunk(p_hbm, src2_hbm, dst2_hbm, out_hbm,
               acc, rows_v, sidx_v, didx_v, sem_g, sem_s, s):
    @pl.when(s < NS - 1)
    def _():
        pltpu.sync_copy(p_hbm.at[pl.ds(s * STRIPE, STRIPE)],
                        acc.at[pl.ds(s * STRIPE, STRIPE)])

    @pl.when(s == NS - 1)
    def _():
        pltpu.sync_copy(p_hbm.at[pl.ds((NS - 1) * STRIPE, N - (NS - 1) * STRIPE)],
                        acc.at[pl.ds((NS - 1) * STRIPE, N - (NS - 1) * STRIPE)])

    def load_idx(bi, buf):
        pltpu.sync_copy(src2_hbm.at[pl.ds(s * NWIN + bi * IB, IB)],
                        sidx_v.at[buf])
        pltpu.sync_copy(dst2_hbm.at[pl.ds(s * NWIN + bi * IB, IB)],
                        didx_v.at[buf])

    def start_gather(w, buf):
        pltpu.async_copy(p_hbm.at[sidx_v.at[(w // IB) % 2, w % IB]],
                         rows_v.at[buf], sem_g)

    def wait_gather(buf):
        pltpu.make_async_copy(p_hbm.at[pl.ds(0, KW)], rows_v.at[buf],
                              sem_g).wait()

    def start_scatter(w, buf):
        pltpu.async_copy(rows_v.at[buf],
                         acc.at[didx_v.at[(w // IB) % 2, w % IB]], sem_s,
                         add=True)

    def wait_scatter(buf):
        pltpu.make_async_copy(p_hbm.at[pl.ds(0, KW)], rows_v.at[buf],
                              sem_s).wait()

    load_idx(0, 0)
    plsc.subcore_barrier()
    for k in range(GD):
        start_gather(k, k)

    def body(g, carry):
        for j in range(NBUF):
            w = g * NBUF + j
            wait_gather(j)

            @pl.when(w >= SD)
            def _():
                wait_scatter((j + NBUF - SD) % NBUF)

            @pl.when((w + GD < NWIN) & ((w + GD) % IB == 0))
            def _():
                load_idx((w + GD) // IB, ((w + GD) // IB) % 2)

            @pl.when(w + GD < NWIN)
            def _():
                start_gather(w + GD, (j + GD) % NBUF)

            start_scatter(w, j)
        return carry

    lax.fori_loop(0, NWIN // NBUF, body, 0)
    for k in range(SD):
        wait_scatter(NBUF - SD + k)
    plsc.subcore_barrier()
    pltpu.sync_copy(acc.at[pl.ds(s * STRIPE, STRIPE)],
                    out_hbm.at[pl.ds(s * STRIPE, STRIPE)])
    plsc.subcore_barrier()


def _agg_body(p0, p1, p2, p3, src2_hbm, dst2_hbm,
              a0, a1, a2, a3,
              acc, rows_v, sidx_v, didx_v, sem_g, sem_s):
    c = lax.axis_index("c")
    s = lax.axis_index("s")

    @pl.when(c == 0)
    def _():
        _agg_chunk(p0, src2_hbm, dst2_hbm, a0, acc, rows_v,
                   sidx_v, didx_v, sem_g, sem_s, s)
        _agg_chunk(p1, src2_hbm, dst2_hbm, a1, acc, rows_v,
                   sidx_v, didx_v, sem_g, sem_s, s)

    @pl.when(c == 1)
    def _():
        _agg_chunk(p2, src2_hbm, dst2_hbm, a2, acc, rows_v,
                   sidx_v, didx_v, sem_g, sem_s, s)
        _agg_chunk(p3, src2_hbm, dst2_hbm, a3, acc, rows_v,
                   sidx_v, didx_v, sem_g, sem_s, s)


_agg_call = pl.kernel(
    _agg_body,
    out_type=[jax.ShapeDtypeStruct((NR, FC), jnp.float32)] * NFC,
    mesh=_mesh,
    scratch_types=[
        pltpu.VMEM_SHARED((NR, FC), jnp.float32),
        pltpu.VMEM((NBUF, KW, FC), jnp.float32),
        pltpu.VMEM((2, IB, KW), jnp.int32),
        pltpu.VMEM((2, IB, KW), jnp.int32),
        pltpu.SemaphoreType.DMA,
        pltpu.SemaphoreType.DMA,
    ],
)


def _mm1_body(x_ref, w_ref, dega_ref, degb_ref, o0, o1, o2, o3):
    dis = lax.rsqrt(dega_ref[0, 0, :] + degb_ref[0, 0, :] + 1.0)
    h = jnp.dot(x_ref[...], w_ref[...], preferred_element_type=jnp.float32)
    p = h * dis[:, None]
    o0[...] = p[:, 0 * FC:1 * FC]
    o1[...] = p[:, 1 * FC:2 * FC]
    o2[...] = p[:, 2 * FC:3 * FC]
    o3[...] = p[:, 3 * FC:4 * FC]


def _mm1(x, W1, dega3, degb3):
    return pl.pallas_call(
        _mm1_body,
        grid=(NB,),
        in_specs=[
            pl.BlockSpec((BN, IN), lambda i: (i, 0)),
            pl.BlockSpec((IN, H), lambda i: (0, 0)),
            pl.BlockSpec((1, 1, BN), lambda i: (i, 0, 0)),
            pl.BlockSpec((1, 1, BN), lambda i: (i, 0, 0)),
        ],
        out_specs=[pl.BlockSpec((BN, FC), lambda i: (i, 0))] * NFC,
        out_shape=[jax.ShapeDtypeStruct((N, FC), jnp.float32)] * NFC,
    )(x, W1, dega3, degb3)


def _mm2_body(a0, a1, a2, a3, dega_ref, degb_ref,
              b_ref, w_ref, o0, o1, o2, o3):
    dis = lax.rsqrt(dega_ref[0, 0, :] + degb_ref[0, 0, :] + 1.0)
    hcat = jnp.concatenate([a0[...], a1[...], a2[...], a3[...]], axis=1)
    h1 = jnp.maximum(hcat * dis[:, None] + b_ref[0, :], 0.0)
    h2 = jnp.dot(h1, w_ref[...], preferred_element_type=jnp.float32)
    p = h2 * dis[:, None]
    o0[...] = p[:, 0 * FC:1 * FC]
    o1[...] = p[:, 1 * FC:2 * FC]
    o2[...] = p[:, 2 * FC:3 * FC]
    o3[...] = p[:, 3 * FC:4 * FC]


def _mm2(aggs, dega3, degb3, b1r, W2):
    return pl.pallas_call(
        _mm2_body,
        grid=(NB,),
        in_specs=(
            [pl.BlockSpec((BN, FC), lambda i: (i, 0))] * NFC
            + [pl.BlockSpec((1, 1, BN), lambda i: (i, 0, 0))] * 2
            + [pl.BlockSpec((1, H), lambda i: (0, 0)),
               pl.BlockSpec((H, H), lambda i: (0, 0))]
        ),
        out_specs=[pl.BlockSpec((BN, FC), lambda i: (i, 0))] * NFC,
        out_shape=[jax.ShapeDtypeStruct((N, FC), jnp.float32)] * NFC,
    )(*aggs, dega3, degb3, b1r, W2)


def _pool_body(a0, a1, a2, a3, dega_ref, degb_ref,
               b_ref, batch_ref, wmu_ref, bmu_ref, wlv_ref, blv_ref,
               wd1_ref, bd1_ref, eps_ref,
               hd_out, mu_out, lv_out, s_scr, c_scr):
    i = pl.program_id(0)

    @pl.when(i == 0)
    def _():
        s_scr[...] = jnp.zeros_like(s_scr)
        c_scr[...] = jnp.zeros_like(c_scr)

    dis = lax.rsqrt(dega_ref[0, 0, :] + degb_ref[0, 0, :] + 1.0)
    hcat = jnp.concatenate([a0[...], a1[...], a2[...], a3[...]], axis=1)
    h2 = jnp.maximum(hcat * dis[:, None] + b_ref[0, :], 0.0)
    bt = batch_ref[0, 0, :]
    oh = (bt[None, :] == lax.broadcasted_iota(jnp.int32, (B, BN), 0)
          ).astype(jnp.float32)
    s_scr[...] += jnp.dot(oh, h2, preferred_element_type=jnp.float32)
    c_scr[...] += jnp.sum(oh, axis=1, keepdims=True)

    @pl.when(i == NB - 1)
    def _():
        g = s_scr[...] / jnp.maximum(c_scr[:, 0:1], 1.0)
        mu = jnp.dot(g, wmu_ref[...], preferred_element_type=jnp.float32) + bmu_ref[0, :]
        lv = jnp.dot(g, wlv_ref[...], preferred_element_type=jnp.float32) + blv_ref[0, :]
        std = jnp.exp(0.5 * lv)
        z = mu + eps_ref[...] * std
        hd = jnp.maximum(
            jnp.dot(z, wd1_ref[...], preferred_element_type=jnp.float32) + bd1_ref[0, :],
            0.0)
        hd_out[...] = hd
        mu_out[...] = mu
        lv_out[...] = lv


def _pool(aggs, dega3, degb3, b2r, batch3, Wmu, bmur, Wlv, blvr,
          Wd1, bd1r, eps):
    full = lambda a, b: pl.BlockSpec((a, b), lambda i: (0, 0))
    return pl.pallas_call(
        _pool_body,
        grid=(NB,),
        in_specs=(
            [pl.BlockSpec((BN, FC), lambda i: (i, 0))] * NFC
            + [pl.BlockSpec((1, 1, BN), lambda i: (i, 0, 0))] * 2
            + [full(1, H), pl.BlockSpec((1, 1, BN), lambda i: (i, 0, 0)),
               full(H, LAT), full(1, LAT), full(H, LAT), full(1, LAT),
               full(LAT, H), full(1, H), full(B, LAT)]
        ),
        out_specs=[full(B, H), full(B, LAT), full(B, LAT)],
        out_shape=[jax.ShapeDtypeStruct((B, H), jnp.float32),
                   jax.ShapeDtypeStruct((B, LAT), jnp.float32),
                   jax.ShapeDtypeStruct((B, LAT), jnp.float32)],
        scratch_shapes=[pltpu.VMEM((B, H), jnp.float32),
                        pltpu.VMEM((B, FC), jnp.float32)],
    )(*aggs, dega3, degb3, b2r, batch3, Wmu, bmur, Wlv, blvr,
      Wd1, bd1r, eps)


_NXC = 8
_DW = MAXN * IN // _NXC


def _dec_body(hd_ref, wn_ref, bn_ref, wa_ref, ba_ref, xr_out, adj_out):
    j = pl.program_id(0)

    @pl.when(j < _NXC)
    def _():
        xr_out[...] = jnp.dot(hd_ref[...], wn_ref[...],
                              preferred_element_type=jnp.float32) + bn_ref[0, :]

    @pl.when(j == _NXC)
    def _():
        a = jnp.dot(hd_ref[...], wa_ref[...],
                    preferred_element_type=jnp.float32) + ba_ref[0, :]
        sg = 1.0 / (1.0 + jnp.exp(-a))
        ci = lax.broadcasted_iota(jnp.int32, (B, MAXN * MAXN), 1)
        diag = (ci // MAXN) == (ci % MAXN)
        adj_out[...] = jnp.where(diag, 0.0, sg)


def _dec(hd, Wn, bnr, Wa, bar):
    cap = lambda j: (0, jnp.minimum(j, _NXC - 1))
    return pl.pallas_call(
        _dec_body,
        grid=(_NXC + 1,),
        in_specs=[
            pl.BlockSpec((B, H), lambda j: (0, 0)),
            pl.BlockSpec((H, _DW), lambda j: cap(j)),
            pl.BlockSpec((1, _DW), lambda j: cap(j)),
            pl.BlockSpec((H, MAXN * MAXN), lambda j: (0, 0)),
            pl.BlockSpec((1, MAXN * MAXN), lambda j: (0, 0)),
        ],
        out_specs=[pl.BlockSpec((B, _DW), lambda j: cap(j)),
                   pl.BlockSpec((B, MAXN * MAXN), lambda j: (0, 0))],
        out_shape=[jax.ShapeDtypeStruct((B, MAXN * IN), jnp.float32),
                   jax.ShapeDtypeStruct((B, MAXN * MAXN), jnp.float32)],
    )(hd, Wn, bnr, Wa, bar)


def kernel(x, edge_index, batch, W1, b1, W2, b2, Wmu, bmu, Wlv, blv,
           Wd1, bd1, Wn, bn, Wa, ba):
    npad = EPAD - E
    fill = jnp.arange(npad, dtype=jnp.int32)
    src_pad = jnp.concatenate([edge_index[0], fill % N])
    dst_pad = jnp.concatenate([edge_index[1], N + fill % (NR - N)])

    zeros_1d = jnp.zeros((NR,), jnp.float32)
    ones_w = jnp.ones((DKW,), jnp.float32)

    dega, degb = _deg_call(dst_pad.reshape(-1, DKW), ones_w, zeros_1d)
    dega3 = dega[:N].reshape(NB, 1, BN)
    degb3 = degb[:N].reshape(NB, 1, BN)

    src2 = src_pad.reshape(-1, KW)
    dst2 = dst_pad.reshape(-1, KW)
    ps1 = _mm1(x, W1, dega3, degb3)
    aggs1 = _agg_call(*ps1, src2, dst2)

    ps2 = _mm2(aggs1, dega3, degb3, b1.reshape(1, H), W2)
    aggs2 = _agg_call(*ps2, src2, dst2)

    eps = jax.random.normal(jax.random.key(42), (B, LAT), jnp.float32)
    batch3 = batch.reshape(NB, 1, BN)
    hd, mu, logvar = _pool(aggs2, dega3, degb3, b2.reshape(1, H),
                           batch3, Wmu, bmu.reshape(1, LAT), Wlv,
                           blv.reshape(1, LAT), Wd1, bd1.reshape(1, H), eps)

    xr, adj = _dec(hd, Wn, bn.reshape(1, MAXN * IN), Wa,
                   ba.reshape(1, MAXN * MAXN))
    return (xr.reshape(B, MAXN, IN), adj.reshape(B, MAXN, MAXN), mu, logvar)


# speedup vs baseline: 1.0564x; 1.0377x over previous
import functools

import jax
import jax.numpy as jnp
from jax import lax
from jax.experimental import pallas as pl
from jax.experimental.pallas import tpu as pltpu
from jax.experimental.pallas import tpu_sc as plsc

N = 10000
E = 160000
IN = 256
H = 512
LAT = 128
MAXN = 32
B = 64

NC = 2
NS = 16
KW = 64
NR = 10240
STRIPE = NR // NS
EPAD = 163840
FC = 128
NFC = H // FC
BN = 400
NB = N // BN

_mesh = plsc.VectorSubcoreMesh(core_axis_name="c", subcore_axis_name="s")


DKW = 128
DWIN = EPAD // (NC * NS) // DKW


def _deg_body(dst128_hbm, ones_hbm, zeros_hbm, dega, degb,
              acc, ones_v, idx_v, sem_s):
    c = lax.axis_index("c")
    s = lax.axis_index("s")
    t = c * NS + s
    pltpu.sync_copy(ones_hbm, ones_v)
    pltpu.sync_copy(zeros_hbm.at[pl.ds(s * STRIPE, STRIPE)],
                    acc.at[pl.ds(s * STRIPE, STRIPE)])
    pltpu.sync_copy(dst128_hbm.at[pl.ds(t * DWIN, DWIN)], idx_v)
    plsc.subcore_barrier()

    def fire(i, carry):
        pltpu.async_copy(ones_v, acc.at[idx_v.at[i]], sem_s, add=True)
        return carry

    lax.fori_loop(0, DWIN, fire, 0)

    def drain(i, carry):
        pltpu.make_async_copy(zeros_hbm.at[pl.ds(0, DKW)], ones_v,
                              sem_s).wait()
        return carry

    lax.fori_loop(0, DWIN, drain, 0)
    plsc.subcore_barrier()

    @pl.when(c == 0)
    def _():
        pltpu.sync_copy(acc.at[pl.ds(s * STRIPE, STRIPE)],
                        dega.at[pl.ds(s * STRIPE, STRIPE)])

    @pl.when(c == 1)
    def _():
        pltpu.sync_copy(acc.at[pl.ds(s * STRIPE, STRIPE)],
                        degb.at[pl.ds(s * STRIPE, STRIPE)])


_deg_call = pl.kernel(
    _deg_body,
    out_type=[jax.ShapeDtypeStruct((NR,), jnp.float32),
              jax.ShapeDtypeStruct((NR,), jnp.float32)],
    mesh=_mesh,
    scratch_types=[
        pltpu.VMEM_SHARED((NR,), jnp.float32),
        pltpu.VMEM((DKW,), jnp.float32),
        pltpu.VMEM((DWIN, DKW), jnp.int32),
        pltpu.SemaphoreType.DMA,
    ],
)


NWIN = EPAD // NS // KW
IB = 16
NBUF = 5
GD = 3
SD = 2


def _agg_chunk(p_hbm, src2_hbm, dst2_hbm, out_hbm,
               acc, rows_v, sidx_v, didx_v, sem_g, sem_s, s):
    @pl.when(s < NS - 1)
    def _():
        pltpu.sync_copy(p_hbm.at[pl.ds(s * STRIPE, STRIPE)],
                        acc.at[pl.ds(s * STRIPE, STRIPE)])

    @pl.when(s == NS - 1)
    def _():
        pltpu.sync_copy(p_hbm.at[pl.ds((NS - 1) * STRIPE, N - (NS - 1) * STRIPE)],
                        acc.at[pl.ds((NS - 1) * STRIPE, N - (NS - 1) * STRIPE)])

    def load_idx(bi, buf):
        pltpu.sync_copy(src2_hbm.at[pl.ds(s * NWIN + bi * IB, IB)],
                        sidx_v.at[buf])
        pltpu.sync_copy(dst2_hbm.at[pl.ds(s * NWIN + bi * IB, IB)],
                        didx_v.at[buf])

    def start_gather(w, buf):
        pltpu.async_copy(p_hbm.at[sidx_v.at[(w // IB) % 2, w % IB]],
                         rows_v.at[buf], sem_g)

    def wait_gather(buf):
        pltpu.make_async_copy(p_hbm.at[pl.ds(0, KW)], rows_v.at[buf],
                              sem_g).wait()

    def start_scatter(w, buf):
        pltpu.async_copy(rows_v.at[buf],
                         acc.at[didx_v.at[(w // IB) % 2, w % IB]], sem_s,
                         add=True)

    def wait_scatter(buf):
        pltpu.make_async_copy(p_hbm.at[pl.ds(0, KW)], rows_v.at[buf],
                              sem_s).wait()

    load_idx(0, 0)
    plsc.subcore_barrier()
    for k in range(GD):
        start_gather(k, k)

    def body(g, carry):
        for j in range(NBUF):
            w = g * NBUF + j
            wait_gather(j)

            @pl.when(w >= SD)
            def _():
                wait_scatter((j + NBUF - SD) % NBUF)

            @pl.when((w + GD < NWIN) & ((w + GD) % IB == 0))
            def _():
                load_idx((w + GD) // IB, ((w + GD) // IB) % 2)

            @pl.when(w + GD < NWIN)
            def _():
                start_gather(w + GD, (j + GD) % NBUF)

            start_scatter(w, j)
        return carry

    lax.fori_loop(0, NWIN // NBUF, body, 0)
    for k in range(SD):
        wait_scatter(NBUF - SD + k)
    plsc.subcore_barrier()
    pltpu.sync_copy(acc.at[pl.ds(s * STRIPE, STRIPE)],
                    out_hbm.at[pl.ds(s * STRIPE, STRIPE)])
    plsc.subcore_barrier()


def _agg_body(p0, p1, p2, p3, src2_hbm, dst2_hbm,
              a0, a1, a2, a3,
              acc, rows_v, sidx_v, didx_v, sem_g, sem_s):
    c = lax.axis_index("c")
    s = lax.axis_index("s")

    @pl.when(c == 0)
    def _():
        _agg_chunk(p0, src2_hbm, dst2_hbm, a0, acc, rows_v,
                   sidx_v, didx_v, sem_g, sem_s, s)
        _agg_chunk(p1, src2_hbm, dst2_hbm, a1, acc, rows_v,
                   sidx_v, didx_v, sem_g, sem_s, s)

    @pl.when(c == 1)
    def _():
        _agg_chunk(p2, src2_hbm, dst2_hbm, a2, acc, rows_v,
                   sidx_v, didx_v, sem_g, sem_s, s)
        _agg_chunk(p3, src2_hbm, dst2_hbm, a3, acc, rows_v,
                   sidx_v, didx_v, sem_g, sem_s, s)


_agg_call = pl.kernel(
    _agg_body,
    out_type=[jax.ShapeDtypeStruct((NR, FC), jnp.float32)] * NFC,
    mesh=_mesh,
    scratch_types=[
        pltpu.VMEM_SHARED((NR, FC), jnp.float32),
        pltpu.VMEM((NBUF, KW, FC), jnp.float32),
        pltpu.VMEM((2, IB, KW), jnp.int32),
        pltpu.VMEM((2, IB, KW), jnp.int32),
        pltpu.SemaphoreType.DMA,
        pltpu.SemaphoreType.DMA,
    ],
)


def _mm1_body(x_ref, w_ref, dega_ref, degb_ref, o0, o1, o2, o3):
    dis = lax.rsqrt(dega_ref[0, 0, :] + degb_ref[0, 0, :] + 1.0)
    h = jnp.dot(x_ref[...], w_ref[...], preferred_element_type=jnp.float32)
    p = h * dis[:, None]
    o0[...] = p[:, 0 * FC:1 * FC]
    o1[...] = p[:, 1 * FC:2 * FC]
    o2[...] = p[:, 2 * FC:3 * FC]
    o3[...] = p[:, 3 * FC:4 * FC]


def _mm1(x, W1, dega3, degb3):
    return pl.pallas_call(
        _mm1_body,
        grid=(NB,),
        in_specs=[
            pl.BlockSpec((BN, IN), lambda i: (i, 0)),
            pl.BlockSpec((IN, H), lambda i: (0, 0)),
            pl.BlockSpec((1, 1, BN), lambda i: (i, 0, 0)),
            pl.BlockSpec((1, 1, BN), lambda i: (i, 0, 0)),
        ],
        out_specs=[pl.BlockSpec((BN, FC), lambda i: (i, 0))] * NFC,
        out_shape=[jax.ShapeDtypeStruct((N, FC), jnp.float32)] * NFC,
    )(x, W1, dega3, degb3)


def _mm2_body(a0, a1, a2, a3, dega_ref, degb_ref,
              b_ref, w_ref, o0, o1, o2, o3):
    dis = lax.rsqrt(dega_ref[0, 0, :] + degb_ref[0, 0, :] + 1.0)
    hcat = jnp.concatenate([a0[...], a1[...], a2[...], a3[...]], axis=1)
    h1 = jnp.maximum(hcat * dis[:, None] + b_ref[0, :], 0.0)
    h2 = jnp.dot(h1, w_ref[...], preferred_element_type=jnp.float32)
    p = h2 * dis[:, None]
    o0[...] = p[:, 0 * FC:1 * FC]
    o1[...] = p[:, 1 * FC:2 * FC]
    o2[...] = p[:, 2 * FC:3 * FC]
    o3[...] = p[:, 3 * FC:4 * FC]


def _mm2(aggs, dega3, degb3, b1r, W2):
    return pl.pallas_call(
        _mm2_body,
        grid=(NB,),
        in_specs=(
            [pl.BlockSpec((BN, FC), lambda i: (i, 0))] * NFC
            + [pl.BlockSpec((1, 1, BN), lambda i: (i, 0, 0))] * 2
            + [pl.BlockSpec((1, H), lambda i: (0, 0)),
               pl.BlockSpec((H, H), lambda i: (0, 0))]
        ),
        out_specs=[pl.BlockSpec((BN, FC), lambda i: (i, 0))] * NFC,
        out_shape=[jax.ShapeDtypeStruct((N, FC), jnp.float32)] * NFC,
    )(*aggs, dega3, degb3, b1r, W2)


_NXC = 8
_DW = MAXN * IN // _NXC


def _tail_body(a0, a1, a2, a3, dega_ref, degb_ref,
               b_ref, batch_ref, wmu_ref, bmu_ref, wlv_ref, blv_ref,
               wd1_ref, bd1_ref, eps_ref, wn_ref, bn_ref, wa_ref, ba_ref,
               xr_out, adj_out, mu_out, lv_out, s_scr, c_scr, hd_scr):
    i = pl.program_id(0)

    @pl.when(i == 0)
    def _():
        s_scr[...] = jnp.zeros_like(s_scr)
        c_scr[...] = jnp.zeros_like(c_scr)

    @pl.when(i < NB)
    def _():
        dis = lax.rsqrt(dega_ref[0, 0, :] + degb_ref[0, 0, :] + 1.0)
        hcat = jnp.concatenate([a0[...], a1[...], a2[...], a3[...]], axis=1)
        h2 = jnp.maximum(hcat * dis[:, None] + b_ref[0, :], 0.0)
        bt = batch_ref[0, 0, :]
        oh = (bt[None, :] == lax.broadcasted_iota(jnp.int32, (B, BN), 0)
              ).astype(jnp.float32)
        s_scr[...] += jnp.dot(oh, h2, preferred_element_type=jnp.float32)
        c_scr[...] += jnp.sum(oh, axis=1, keepdims=True)

    @pl.when(i == NB - 1)
    def _():
        g = s_scr[...] / jnp.maximum(c_scr[:, 0:1], 1.0)
        mu = jnp.dot(g, wmu_ref[...], preferred_element_type=jnp.float32) + bmu_ref[0, :]
        lv = jnp.dot(g, wlv_ref[...], preferred_element_type=jnp.float32) + blv_ref[0, :]
        std = jnp.exp(0.5 * lv)
        z = mu + eps_ref[...] * std
        hd_scr[...] = jnp.maximum(
            jnp.dot(z, wd1_ref[...], preferred_element_type=jnp.float32) + bd1_ref[0, :],
            0.0)
        mu_out[...] = mu
        lv_out[...] = lv

    @pl.when((i >= NB) & (i < NB + _NXC))
    def _():
        xr_out[...] = jnp.dot(hd_scr[...], wn_ref[...],
                              preferred_element_type=jnp.float32) + bn_ref[0, :]

    @pl.when(i == NB + _NXC)
    def _():
        a = jnp.dot(hd_scr[...], wa_ref[...],
                    preferred_element_type=jnp.float32) + ba_ref[0, :]
        sg = 1.0 / (1.0 + jnp.exp(-a))
        ci = lax.broadcasted_iota(jnp.int32, (B, MAXN * MAXN), 1)
        diag = (ci // MAXN) == (ci % MAXN)
        adj_out[...] = jnp.where(diag, 0.0, sg)


def _tail(aggs, dega3, degb3, b2r, batch3, Wmu, bmur, Wlv, blvr,
          Wd1, bd1r, eps, Wn, bnr, Wa, bar):
    full = lambda a, b: pl.BlockSpec((a, b), lambda i: (0, 0))
    nblk = lambda i: (jnp.minimum(i, NB - 1), 0)
    nblk3 = lambda i: (jnp.minimum(i, NB - 1), 0, 0)
    dblk = lambda i: (0, jnp.clip(i - NB, 0, _NXC - 1))
    return pl.pallas_call(
        _tail_body,
        grid=(NB + _NXC + 1,),
        in_specs=(
            [pl.BlockSpec((BN, FC), nblk)] * NFC
            + [pl.BlockSpec((1, 1, BN), nblk3)] * 2
            + [full(1, H), pl.BlockSpec((1, 1, BN), nblk3),
               full(H, LAT), full(1, LAT), full(H, LAT), full(1, LAT),
               full(LAT, H), full(1, H), full(B, LAT),
               pl.BlockSpec((H, _DW), dblk), pl.BlockSpec((1, _DW), dblk),
               full(H, MAXN * MAXN), full(1, MAXN * MAXN)]
        ),
        out_specs=[pl.BlockSpec((B, _DW), dblk),
                   full(B, MAXN * MAXN), full(B, LAT), full(B, LAT)],
        out_shape=[jax.ShapeDtypeStruct((B, MAXN * IN), jnp.float32),
                   jax.ShapeDtypeStruct((B, MAXN * MAXN), jnp.float32),
                   jax.ShapeDtypeStruct((B, LAT), jnp.float32),
                   jax.ShapeDtypeStruct((B, LAT), jnp.float32)],
        scratch_shapes=[pltpu.VMEM((B, H), jnp.float32),
                        pltpu.VMEM((B, FC), jnp.float32),
                        pltpu.VMEM((B, H), jnp.float32)],
    )(*aggs, dega3, degb3, b2r, batch3, Wmu, bmur, Wlv, blvr,
      Wd1, bd1r, eps, Wn, bnr, Wa, bar)


def kernel(x, edge_index, batch, W1, b1, W2, b2, Wmu, bmu, Wlv, blv,
           Wd1, bd1, Wn, bn, Wa, ba):
    npad = EPAD - E
    fill = jnp.arange(npad, dtype=jnp.int32)
    src_pad = jnp.concatenate([edge_index[0], fill % N])
    dst_pad = jnp.concatenate([edge_index[1], N + fill % (NR - N)])

    zeros_1d = jnp.zeros((NR,), jnp.float32)
    ones_w = jnp.ones((DKW,), jnp.float32)

    dega, degb = _deg_call(dst_pad.reshape(-1, DKW), ones_w, zeros_1d)
    dega3 = dega[:N].reshape(NB, 1, BN)
    degb3 = degb[:N].reshape(NB, 1, BN)

    src2 = src_pad.reshape(-1, KW)
    dst2 = dst_pad.reshape(-1, KW)
    ps1 = _mm1(x, W1, dega3, degb3)
    aggs1 = _agg_call(*ps1, src2, dst2)

    ps2 = _mm2(aggs1, dega3, degb3, b1.reshape(1, H), W2)
    aggs2 = _agg_call(*ps2, src2, dst2)

    eps = jax.random.normal(jax.random.key(42), (B, LAT), jnp.float32)
    batch3 = batch.reshape(NB, 1, BN)
    xr, adj, mu, logvar = _tail(
        aggs2, dega3, degb3, b2.reshape(1, H), batch3,
        Wmu, bmu.reshape(1, LAT), Wlv, blv.reshape(1, LAT),
        Wd1, bd1.reshape(1, H), eps, Wn, bn.reshape(1, MAXN * IN),
        Wa, ba.reshape(1, MAXN * MAXN))
    return (xr.reshape(B, MAXN, IN), adj.reshape(B, MAXN, MAXN), mu, logvar)


# speedup vs baseline: 1.0615x; 1.0048x over previous
import functools

import jax
import jax.numpy as jnp
from jax import lax
from jax.experimental import pallas as pl
from jax.experimental.pallas import tpu as pltpu
from jax.experimental.pallas import tpu_sc as plsc

N = 10000
E = 160000
IN = 256
H = 512
LAT = 128
MAXN = 32
B = 64

NC = 2
NS = 16
KW = 64
NR = 10240
STRIPE = NR // NS
EPAD = 163840
FC = 128
NFC = H // FC
BN = 400
NB = N // BN

_mesh = plsc.VectorSubcoreMesh(core_axis_name="c", subcore_axis_name="s")


DKW = 128
DWIN = EPAD // (NC * NS) // DKW


def _deg_body(dst128_hbm, ones_hbm, zeros_hbm, dega, degb,
              acc, ones_v, idx_v, sem_s):
    c = lax.axis_index("c")
    s = lax.axis_index("s")
    t = c * NS + s
    pltpu.sync_copy(ones_hbm, ones_v)
    pltpu.sync_copy(zeros_hbm.at[pl.ds(s * STRIPE, STRIPE)],
                    acc.at[pl.ds(s * STRIPE, STRIPE)])
    pltpu.sync_copy(dst128_hbm.at[pl.ds(t * DWIN, DWIN)], idx_v)
    plsc.subcore_barrier()

    def fire(i, carry):
        pltpu.async_copy(ones_v, acc.at[idx_v.at[i]], sem_s, add=True)
        return carry

    lax.fori_loop(0, DWIN, fire, 0)

    def drain(i, carry):
        pltpu.make_async_copy(zeros_hbm.at[pl.ds(0, DKW)], ones_v,
                              sem_s).wait()
        return carry

    lax.fori_loop(0, DWIN, drain, 0)
    plsc.subcore_barrier()

    @pl.when(c == 0)
    def _():
        pltpu.sync_copy(acc.at[pl.ds(s * STRIPE, STRIPE)],
                        dega.at[pl.ds(s * STRIPE, STRIPE)])

    @pl.when(c == 1)
    def _():
        pltpu.sync_copy(acc.at[pl.ds(s * STRIPE, STRIPE)],
                        degb.at[pl.ds(s * STRIPE, STRIPE)])


_deg_call = pl.kernel(
    _deg_body,
    out_type=[jax.ShapeDtypeStruct((NR,), jnp.float32),
              jax.ShapeDtypeStruct((NR,), jnp.float32)],
    mesh=_mesh,
    scratch_types=[
        pltpu.VMEM_SHARED((NR,), jnp.float32),
        pltpu.VMEM((DKW,), jnp.float32),
        pltpu.VMEM((DWIN, DKW), jnp.int32),
        pltpu.SemaphoreType.DMA,
    ],
)


NWIN = EPAD // NS // KW
IB = 16
NBUF = 5
GD = 4
SD = 1


def _agg_chunk(p_hbm, src2_hbm, dst2_hbm, out_hbm,
               acc, rows_v, sidx_v, didx_v, sem_g, sem_s, s):
    @pl.when(s < NS - 1)
    def _():
        pltpu.sync_copy(p_hbm.at[pl.ds(s * STRIPE, STRIPE)],
                        acc.at[pl.ds(s * STRIPE, STRIPE)])

    @pl.when(s == NS - 1)
    def _():
        pltpu.sync_copy(p_hbm.at[pl.ds((NS - 1) * STRIPE, N - (NS - 1) * STRIPE)],
                        acc.at[pl.ds((NS - 1) * STRIPE, N - (NS - 1) * STRIPE)])

    def load_idx(bi, buf):
        pltpu.sync_copy(src2_hbm.at[pl.ds(s * NWIN + bi * IB, IB)],
                        sidx_v.at[buf])
        pltpu.sync_copy(dst2_hbm.at[pl.ds(s * NWIN + bi * IB, IB)],
                        didx_v.at[buf])

    def start_gather(w, buf):
        pltpu.async_copy(p_hbm.at[sidx_v.at[(w // IB) % 2, w % IB]],
                         rows_v.at[buf], sem_g)

    def wait_gather(buf):
        pltpu.make_async_copy(p_hbm.at[pl.ds(0, KW)], rows_v.at[buf],
                              sem_g).wait()

    def start_scatter(w, buf):
        pltpu.async_copy(rows_v.at[buf],
                         acc.at[didx_v.at[(w // IB) % 2, w % IB]], sem_s,
                         add=True)

    def wait_scatter(buf):
        pltpu.make_async_copy(p_hbm.at[pl.ds(0, KW)], rows_v.at[buf],
                              sem_s).wait()

    load_idx(0, 0)
    plsc.subcore_barrier()
    for k in range(GD):
        start_gather(k, k)

    def body(g, carry):
        for j in range(NBUF):
            w = g * NBUF + j
            wait_gather(j)

            @pl.when(w >= SD)
            def _():
                wait_scatter((j + NBUF - SD) % NBUF)

            @pl.when((w + GD < NWIN) & ((w + GD) % IB == 0))
            def _():
                load_idx((w + GD) // IB, ((w + GD) // IB) % 2)

            @pl.when(w + GD < NWIN)
            def _():
                start_gather(w + GD, (j + GD) % NBUF)

            start_scatter(w, j)
        return carry

    lax.fori_loop(0, NWIN // NBUF, body, 0)
    for k in range(SD):
        wait_scatter(NBUF - SD + k)
    plsc.subcore_barrier()
    pltpu.sync_copy(acc.at[pl.ds(s * STRIPE, STRIPE)],
                    out_hbm.at[pl.ds(s * STRIPE, STRIPE)])
    plsc.subcore_barrier()


def _agg_body(p0, p1, p2, p3, src2_hbm, dst2_hbm,
              a0, a1, a2, a3,
              acc, rows_v, sidx_v, didx_v, sem_g, sem_s):
    c = lax.axis_index("c")
    s = lax.axis_index("s")

    @pl.when(c == 0)
    def _():
        _agg_chunk(p0, src2_hbm, dst2_hbm, a0, acc, rows_v,
                   sidx_v, didx_v, sem_g, sem_s, s)
        _agg_chunk(p1, src2_hbm, dst2_hbm, a1, acc, rows_v,
                   sidx_v, didx_v, sem_g, sem_s, s)

    @pl.when(c == 1)
    def _():
        _agg_chunk(p2, src2_hbm, dst2_hbm, a2, acc, rows_v,
                   sidx_v, didx_v, sem_g, sem_s, s)
        _agg_chunk(p3, src2_hbm, dst2_hbm, a3, acc, rows_v,
                   sidx_v, didx_v, sem_g, sem_s, s)


_agg_call = pl.kernel(
    _agg_body,
    out_type=[jax.ShapeDtypeStruct((NR, FC), jnp.float32)] * NFC,
    mesh=_mesh,
    scratch_types=[
        pltpu.VMEM_SHARED((NR, FC), jnp.float32),
        pltpu.VMEM((NBUF, KW, FC), jnp.float32),
        pltpu.VMEM((2, IB, KW), jnp.int32),
        pltpu.VMEM((2, IB, KW), jnp.int32),
        pltpu.SemaphoreType.DMA,
        pltpu.SemaphoreType.DMA,
    ],
)


def _mm1_body(x_ref, w_ref, dega_ref, degb_ref, o0, o1, o2, o3):
    dis = lax.rsqrt(dega_ref[0, 0, :] + degb_ref[0, 0, :] + 1.0)
    h = jnp.dot(x_ref[...], w_ref[...], preferred_element_type=jnp.float32)
    p = h * dis[:, None]
    o0[...] = p[:, 0 * FC:1 * FC]
    o1[...] = p[:, 1 * FC:2 * FC]
    o2[...] = p[:, 2 * FC:3 * FC]
    o3[...] = p[:, 3 * FC:4 * FC]


def _mm1(x, W1, dega3, degb3):
    return pl.pallas_call(
        _mm1_body,
        grid=(NB,),
        in_specs=[
            pl.BlockSpec((BN, IN), lambda i: (i, 0)),
            pl.BlockSpec((IN, H), lambda i: (0, 0)),
            pl.BlockSpec((1, 1, BN), lambda i: (i, 0, 0)),
            pl.BlockSpec((1, 1, BN), lambda i: (i, 0, 0)),
        ],
        out_specs=[pl.BlockSpec((BN, FC), lambda i: (i, 0))] * NFC,
        out_shape=[jax.ShapeDtypeStruct((N, FC), jnp.float32)] * NFC,
    )(x, W1, dega3, degb3)


def _mm2_body(a0, a1, a2, a3, dega_ref, degb_ref,
              b_ref, w_ref, o0, o1, o2, o3):
    dis = lax.rsqrt(dega_ref[0, 0, :] + degb_ref[0, 0, :] + 1.0)
    hcat = jnp.concatenate([a0[...], a1[...], a2[...], a3[...]], axis=1)
    h1 = jnp.maximum(hcat * dis[:, None] + b_ref[0, :], 0.0)
    h2 = jnp.dot(h1, w_ref[...], preferred_element_type=jnp.float32)
    p = h2 * dis[:, None]
    o0[...] = p[:, 0 * FC:1 * FC]
    o1[...] = p[:, 1 * FC:2 * FC]
    o2[...] = p[:, 2 * FC:3 * FC]
    o3[...] = p[:, 3 * FC:4 * FC]


def _mm2(aggs, dega3, degb3, b1r, W2):
    return pl.pallas_call(
        _mm2_body,
        grid=(NB,),
        in_specs=(
            [pl.BlockSpec((BN, FC), lambda i: (i, 0))] * NFC
            + [pl.BlockSpec((1, 1, BN), lambda i: (i, 0, 0))] * 2
            + [pl.BlockSpec((1, H), lambda i: (0, 0)),
               pl.BlockSpec((H, H), lambda i: (0, 0))]
        ),
        out_specs=[pl.BlockSpec((BN, FC), lambda i: (i, 0))] * NFC,
        out_shape=[jax.ShapeDtypeStruct((N, FC), jnp.float32)] * NFC,
    )(*aggs, dega3, degb3, b1r, W2)


_NXC = 8
_DW = MAXN * IN // _NXC


def _tail_body(a0, a1, a2, a3, dega_ref, degb_ref,
               b_ref, batch_ref, wmu_ref, bmu_ref, wlv_ref, blv_ref,
               wd1_ref, bd1_ref, eps_ref, wn_ref, bn_ref, wa_ref, ba_ref,
               xr_out, adj_out, mu_out, lv_out, s_scr, c_scr, hd_scr):
    i = pl.program_id(0)

    @pl.when(i == 0)
    def _():
        s_scr[...] = jnp.zeros_like(s_scr)
        c_scr[...] = jnp.zeros_like(c_scr)

    @pl.when(i < NB)
    def _():
        dis = lax.rsqrt(dega_ref[0, 0, :] + degb_ref[0, 0, :] + 1.0)
        hcat = jnp.concatenate([a0[...], a1[...], a2[...], a3[...]], axis=1)
        h2 = jnp.maximum(hcat * dis[:, None] + b_ref[0, :], 0.0)
        bt = batch_ref[0, 0, :]
        oh = (bt[None, :] == lax.broadcasted_iota(jnp.int32, (B, BN), 0)
              ).astype(jnp.float32)
        s_scr[...] += jnp.dot(oh, h2, preferred_element_type=jnp.float32)
        c_scr[...] += jnp.sum(oh, axis=1, keepdims=True)

    @pl.when(i == NB - 1)
    def _():
        g = s_scr[...] / jnp.maximum(c_scr[:, 0:1], 1.0)
        mu = jnp.dot(g, wmu_ref[...], preferred_element_type=jnp.float32) + bmu_ref[0, :]
        lv = jnp.dot(g, wlv_ref[...], preferred_element_type=jnp.float32) + blv_ref[0, :]
        std = jnp.exp(0.5 * lv)
        z = mu + eps_ref[...] * std
        hd_scr[...] = jnp.maximum(
            jnp.dot(z, wd1_ref[...], preferred_element_type=jnp.float32) + bd1_ref[0, :],
            0.0)
        mu_out[...] = mu
        lv_out[...] = lv

    @pl.when((i >= NB) & (i < NB + _NXC))
    def _():
        xr_out[...] = jnp.dot(hd_scr[...], wn_ref[...],
                              preferred_element_type=jnp.float32) + bn_ref[0, :]

    @pl.when(i == NB + _NXC)
    def _():
        a = jnp.dot(hd_scr[...], wa_ref[...],
                    preferred_element_type=jnp.float32) + ba_ref[0, :]
        sg = 1.0 / (1.0 + jnp.exp(-a))
        ci = lax.broadcasted_iota(jnp.int32, (B, MAXN * MAXN), 1)
        diag = (ci // MAXN) == (ci % MAXN)
        adj_out[...] = jnp.where(diag, 0.0, sg)


def _tail(aggs, dega3, degb3, b2r, batch3, Wmu, bmur, Wlv, blvr,
          Wd1, bd1r, eps, Wn, bnr, Wa, bar):
    full = lambda a, b: pl.BlockSpec((a, b), lambda i: (0, 0))
    nblk = lambda i: (jnp.minimum(i, NB - 1), 0)
    nblk3 = lambda i: (jnp.minimum(i, NB - 1), 0, 0)
    dblk = lambda i: (0, jnp.clip(i - NB, 0, _NXC - 1))
    return pl.pallas_call(
        _tail_body,
        grid=(NB + _NXC + 1,),
        in_specs=(
            [pl.BlockSpec((BN, FC), nblk)] * NFC
            + [pl.BlockSpec((1, 1, BN), nblk3)] * 2
            + [full(1, H), pl.BlockSpec((1, 1, BN), nblk3),
               full(H, LAT), full(1, LAT), full(H, LAT), full(1, LAT),
               full(LAT, H), full(1, H), full(B, LAT),
               pl.BlockSpec((H, _DW), dblk), pl.BlockSpec((1, _DW), dblk),
               full(H, MAXN * MAXN), full(1, MAXN * MAXN)]
        ),
        out_specs=[pl.BlockSpec((B, _DW), dblk),
                   full(B, MAXN * MAXN), full(B, LAT), full(B, LAT)],
        out_shape=[jax.ShapeDtypeStruct((B, MAXN * IN), jnp.float32),
                   jax.ShapeDtypeStruct((B, MAXN * MAXN), jnp.float32),
                   jax.ShapeDtypeStruct((B, LAT), jnp.float32),
                   jax.ShapeDtypeStruct((B, LAT), jnp.float32)],
        scratch_shapes=[pltpu.VMEM((B, H), jnp.float32),
                        pltpu.VMEM((B, FC), jnp.float32),
                        pltpu.VMEM((B, H), jnp.float32)],
    )(*aggs, dega3, degb3, b2r, batch3, Wmu, bmur, Wlv, blvr,
      Wd1, bd1r, eps, Wn, bnr, Wa, bar)


def kernel(x, edge_index, batch, W1, b1, W2, b2, Wmu, bmu, Wlv, blv,
           Wd1, bd1, Wn, bn, Wa, ba):
    npad = EPAD - E
    fill = jnp.arange(npad, dtype=jnp.int32)
    src_pad = jnp.concatenate([edge_index[0], fill % N])
    dst_pad = jnp.concatenate([edge_index[1], N + fill % (NR - N)])

    zeros_1d = jnp.zeros((NR,), jnp.float32)
    ones_w = jnp.ones((DKW,), jnp.float32)

    dega, degb = _deg_call(dst_pad.reshape(-1, DKW), ones_w, zeros_1d)
    dega3 = dega[:N].reshape(NB, 1, BN)
    degb3 = degb[:N].reshape(NB, 1, BN)

    src2 = src_pad.reshape(-1, KW)
    dst2 = dst_pad.reshape(-1, KW)
    ps1 = _mm1(x, W1, dega3, degb3)
    aggs1 = _agg_call(*ps1, src2, dst2)

    ps2 = _mm2(aggs1, dega3, degb3, b1.reshape(1, H), W2)
    aggs2 = _agg_call(*ps2, src2, dst2)

    eps = jax.random.normal(jax.random.key(42), (B, LAT), jnp.float32)
    batch3 = batch.reshape(NB, 1, BN)
    xr, adj, mu, logvar = _tail(
        aggs2, dega3, degb3, b2.reshape(1, H), batch3,
        Wmu, bmu.reshape(1, LAT), Wlv, blv.reshape(1, LAT),
        Wd1, bd1.reshape(1, H), eps, Wn, bn.reshape(1, MAXN * IN),
        Wa, ba.reshape(1, MAXN * MAXN))
    return (xr.reshape(B, MAXN, IN), adj.reshape(B, MAXN, MAXN), mu, logvar)


# speedup vs baseline: 1.0636x; 1.0020x over previous
import functools

import jax
import jax.numpy as jnp
import numpy as np
from jax import lax
from jax.experimental import pallas as pl
from jax.experimental.pallas import tpu as pltpu
from jax.experimental.pallas import tpu_sc as plsc

N = 10000
E = 160000
IN = 256
H = 512
LAT = 128
MAXN = 32
B = 64

NC = 2
NS = 16
KW = 64
NR = 10240
STRIPE = NR // NS
EPAD = 163840
FC = 128
NFC = H // FC
BN = 400
NB = N // BN

_mesh = plsc.VectorSubcoreMesh(core_axis_name="c", subcore_axis_name="s")

_EPS = np.asarray(
    jax.random.normal(jax.random.key(42), (B, LAT), jnp.float32))


DKW = 128
DWIN = EPAD // (NC * NS) // DKW


def _deg_body(dst128_hbm, ones_hbm, zeros_hbm, dega, degb,
              acc, ones_v, idx_v, sem_s):
    c = lax.axis_index("c")
    s = lax.axis_index("s")
    t = c * NS + s
    pltpu.sync_copy(ones_hbm, ones_v)
    pltpu.sync_copy(zeros_hbm.at[pl.ds(s * STRIPE, STRIPE)],
                    acc.at[pl.ds(s * STRIPE, STRIPE)])
    pltpu.sync_copy(dst128_hbm.at[pl.ds(t * DWIN, DWIN)], idx_v)
    plsc.subcore_barrier()

    def fire(i, carry):
        pltpu.async_copy(ones_v, acc.at[idx_v.at[i]], sem_s, add=True)
        return carry

    lax.fori_loop(0, DWIN, fire, 0)

    def drain(i, carry):
        pltpu.make_async_copy(zeros_hbm.at[pl.ds(0, DKW)], ones_v,
                              sem_s).wait()
        return carry

    lax.fori_loop(0, DWIN, drain, 0)
    plsc.subcore_barrier()

    @pl.when(c == 0)
    def _():
        pltpu.sync_copy(acc.at[pl.ds(s * STRIPE, STRIPE)],
                        dega.at[pl.ds(s * STRIPE, STRIPE)])

    @pl.when(c == 1)
    def _():
        pltpu.sync_copy(acc.at[pl.ds(s * STRIPE, STRIPE)],
                        degb.at[pl.ds(s * STRIPE, STRIPE)])


_deg_call = pl.kernel(
    _deg_body,
    out_type=[jax.ShapeDtypeStruct((NR,), jnp.float32),
              jax.ShapeDtypeStruct((NR,), jnp.float32)],
    mesh=_mesh,
    scratch_types=[
        pltpu.VMEM_SHARED((NR,), jnp.float32),
        pltpu.VMEM((DKW,), jnp.float32),
        pltpu.VMEM((DWIN, DKW), jnp.int32),
        pltpu.SemaphoreType.DMA,
    ],
)


NWIN = EPAD // NS // KW
IB = 16
NBUF = 5
GD = 4
SD = 1


def _agg_chunk(p_hbm, src2_hbm, dst2_hbm, out_hbm,
               acc, rows_v, sidx_v, didx_v, sem_g, sem_s, s):
    @pl.when(s < NS - 1)
    def _():
        pltpu.sync_copy(p_hbm.at[pl.ds(s * STRIPE, STRIPE)],
                        acc.at[pl.ds(s * STRIPE, STRIPE)])

    @pl.when(s == NS - 1)
    def _():
        pltpu.sync_copy(p_hbm.at[pl.ds((NS - 1) * STRIPE, N - (NS - 1) * STRIPE)],
                        acc.at[pl.ds((NS - 1) * STRIPE, N - (NS - 1) * STRIPE)])

    def load_idx(bi, buf):
        pltpu.sync_copy(src2_hbm.at[pl.ds(s * NWIN + bi * IB, IB)],
                        sidx_v.at[buf])
        pltpu.sync_copy(dst2_hbm.at[pl.ds(s * NWIN + bi * IB, IB)],
                        didx_v.at[buf])

    def start_gather(w, buf):
        pltpu.async_copy(p_hbm.at[sidx_v.at[(w // IB) % 2, w % IB]],
                         rows_v.at[buf], sem_g)

    def wait_gather(buf):
        pltpu.make_async_copy(p_hbm.at[pl.ds(0, KW)], rows_v.at[buf],
                              sem_g).wait()

    def start_scatter(w, buf):
        pltpu.async_copy(rows_v.at[buf],
                         acc.at[didx_v.at[(w // IB) % 2, w % IB]], sem_s,
                         add=True)

    def wait_scatter(buf):
        pltpu.make_async_copy(p_hbm.at[pl.ds(0, KW)], rows_v.at[buf],
                              sem_s).wait()

    load_idx(0, 0)
    plsc.subcore_barrier()
    for k in range(GD):
        start_gather(k, k)

    def body(g, carry):
        for j in range(NBUF):
            w = g * NBUF + j
            wait_gather(j)

            @pl.when(w >= SD)
            def _():
                wait_scatter((j + NBUF - SD) % NBUF)

            @pl.when((w + GD < NWIN) & ((w + GD) % IB == 0))
            def _():
                load_idx((w + GD) // IB, ((w + GD) // IB) % 2)

            @pl.when(w + GD < NWIN)
            def _():
                start_gather(w + GD, (j + GD) % NBUF)

            start_scatter(w, j)
        return carry

    lax.fori_loop(0, NWIN // NBUF, body, 0)
    for k in range(SD):
        wait_scatter(NBUF - SD + k)
    plsc.subcore_barrier()
    pltpu.sync_copy(acc.at[pl.ds(s * STRIPE, STRIPE)],
                    out_hbm.at[pl.ds(s * STRIPE, STRIPE)])
    plsc.subcore_barrier()


def _agg_body(p0, p1, p2, p3, src2_hbm, dst2_hbm,
              a0, a1, a2, a3,
              acc, rows_v, sidx_v, didx_v, sem_g, sem_s):
    c = lax.axis_index("c")
    s = lax.axis_index("s")

    @pl.when(c == 0)
    def _():
        _agg_chunk(p0, src2_hbm, dst2_hbm, a0, acc, rows_v,
                   sidx_v, didx_v, sem_g, sem_s, s)
        _agg_chunk(p1, src2_hbm, dst2_hbm, a1, acc, rows_v,
                   sidx_v, didx_v, sem_g, sem_s, s)

    @pl.when(c == 1)
    def _():
        _agg_chunk(p2, src2_hbm, dst2_hbm, a2, acc, rows_v,
                   sidx_v, didx_v, sem_g, sem_s, s)
        _agg_chunk(p3, src2_hbm, dst2_hbm, a3, acc, rows_v,
                   sidx_v, didx_v, sem_g, sem_s, s)


_agg_call = pl.kernel(
    _agg_body,
    out_type=[jax.ShapeDtypeStruct((NR, FC), jnp.float32)] * NFC,
    mesh=_mesh,
    scratch_types=[
        pltpu.VMEM_SHARED((NR, FC), jnp.float32),
        pltpu.VMEM((NBUF, KW, FC), jnp.float32),
        pltpu.VMEM((2, IB, KW), jnp.int32),
        pltpu.VMEM((2, IB, KW), jnp.int32),
        pltpu.SemaphoreType.DMA,
        pltpu.SemaphoreType.DMA,
    ],
)


def _mm1_body(x_ref, w_ref, dega_ref, degb_ref, o0, o1, o2, o3):
    dis = lax.rsqrt(dega_ref[0, 0, :] + degb_ref[0, 0, :] + 1.0)
    h = jnp.dot(x_ref[...], w_ref[...], preferred_element_type=jnp.float32)
    p = h * dis[:, None]
    o0[...] = p[:, 0 * FC:1 * FC]
    o1[...] = p[:, 1 * FC:2 * FC]
    o2[...] = p[:, 2 * FC:3 * FC]
    o3[...] = p[:, 3 * FC:4 * FC]


def _mm1(x, W1, dega3, degb3):
    return pl.pallas_call(
        _mm1_body,
        grid=(NB,),
        in_specs=[
            pl.BlockSpec((BN, IN), lambda i: (i, 0)),
            pl.BlockSpec((IN, H), lambda i: (0, 0)),
            pl.BlockSpec((1, 1, BN), lambda i: (i, 0, 0)),
            pl.BlockSpec((1, 1, BN), lambda i: (i, 0, 0)),
        ],
        out_specs=[pl.BlockSpec((BN, FC), lambda i: (i, 0))] * NFC,
        out_shape=[jax.ShapeDtypeStruct((N, FC), jnp.float32)] * NFC,
    )(x, W1, dega3, degb3)


def _mm2_body(a0, a1, a2, a3, dega_ref, degb_ref,
              b_ref, w_ref, o0, o1, o2, o3):
    dis = lax.rsqrt(dega_ref[0, 0, :] + degb_ref[0, 0, :] + 1.0)
    hcat = jnp.concatenate([a0[...], a1[...], a2[...], a3[...]], axis=1)
    h1 = jnp.maximum(hcat * dis[:, None] + b_ref[0, :], 0.0)
    h2 = jnp.dot(h1, w_ref[...], preferred_element_type=jnp.float32)
    p = h2 * dis[:, None]
    o0[...] = p[:, 0 * FC:1 * FC]
    o1[...] = p[:, 1 * FC:2 * FC]
    o2[...] = p[:, 2 * FC:3 * FC]
    o3[...] = p[:, 3 * FC:4 * FC]


def _mm2(aggs, dega3, degb3, b1r, W2):
    return pl.pallas_call(
        _mm2_body,
        grid=(NB,),
        in_specs=(
            [pl.BlockSpec((BN, FC), lambda i: (i, 0))] * NFC
            + [pl.BlockSpec((1, 1, BN), lambda i: (i, 0, 0))] * 2
            + [pl.BlockSpec((1, H), lambda i: (0, 0)),
               pl.BlockSpec((H, H), lambda i: (0, 0))]
        ),
        out_specs=[pl.BlockSpec((BN, FC), lambda i: (i, 0))] * NFC,
        out_shape=[jax.ShapeDtypeStruct((N, FC), jnp.float32)] * NFC,
    )(*aggs, dega3, degb3, b1r, W2)


_NXC = 8
_DW = MAXN * IN // _NXC


def _tail_body(a0, a1, a2, a3, dega_ref, degb_ref,
               b_ref, batch_ref, wmu_ref, bmu_ref, wlv_ref, blv_ref,
               wd1_ref, bd1_ref, eps_ref, wn_ref, bn_ref, wa_ref, ba_ref,
               xr_out, adj_out, mu_out, lv_out, s_scr, c_scr, hd_scr):
    i = pl.program_id(0)

    @pl.when(i == 0)
    def _():
        s_scr[...] = jnp.zeros_like(s_scr)
        c_scr[...] = jnp.zeros_like(c_scr)

    @pl.when(i < NB)
    def _():
        dis = lax.rsqrt(dega_ref[0, 0, :] + degb_ref[0, 0, :] + 1.0)
        hcat = jnp.concatenate([a0[...], a1[...], a2[...], a3[...]], axis=1)
        h2 = jnp.maximum(hcat * dis[:, None] + b_ref[0, :], 0.0)
        bt = batch_ref[0, 0, :]
        oh = (bt[None, :] == lax.broadcasted_iota(jnp.int32, (B, BN), 0)
              ).astype(jnp.float32)
        s_scr[...] += jnp.dot(oh, h2, preferred_element_type=jnp.float32)
        c_scr[...] += jnp.sum(oh, axis=1, keepdims=True)

    @pl.when(i == NB - 1)
    def _():
        g = s_scr[...] / jnp.maximum(c_scr[:, 0:1], 1.0)
        mu = jnp.dot(g, wmu_ref[...], preferred_element_type=jnp.float32) + bmu_ref[0, :]
        lv = jnp.dot(g, wlv_ref[...], preferred_element_type=jnp.float32) + blv_ref[0, :]
        std = jnp.exp(0.5 * lv)
        z = mu + eps_ref[...] * std
        hd_scr[...] = jnp.maximum(
            jnp.dot(z, wd1_ref[...], preferred_element_type=jnp.float32) + bd1_ref[0, :],
            0.0)
        mu_out[...] = mu
        lv_out[...] = lv

    @pl.when((i >= NB) & (i < NB + _NXC))
    def _():
        xr_out[...] = jnp.dot(hd_scr[...], wn_ref[...],
                              preferred_element_type=jnp.float32) + bn_ref[0, :]

    @pl.when(i == NB + _NXC)
    def _():
        a = jnp.dot(hd_scr[...], wa_ref[...],
                    preferred_element_type=jnp.float32) + ba_ref[0, :]
        sg = 1.0 / (1.0 + jnp.exp(-a))
        ci = lax.broadcasted_iota(jnp.int32, (B, MAXN * MAXN), 1)
        diag = (ci // MAXN) == (ci % MAXN)
        adj_out[...] = jnp.where(diag, 0.0, sg)


def _tail(aggs, dega3, degb3, b2r, batch3, Wmu, bmur, Wlv, blvr,
          Wd1, bd1r, eps, Wn, bnr, Wa, bar):
    full = lambda a, b: pl.BlockSpec((a, b), lambda i: (0, 0))
    nblk = lambda i: (jnp.minimum(i, NB - 1), 0)
    nblk3 = lambda i: (jnp.minimum(i, NB - 1), 0, 0)
    dblk = lambda i: (0, jnp.clip(i - NB, 0, _NXC - 1))
    return pl.pallas_call(
        _tail_body,
        grid=(NB + _NXC + 1,),
        in_specs=(
            [pl.BlockSpec((BN, FC), nblk)] * NFC
            + [pl.BlockSpec((1, 1, BN), nblk3)] * 2
            + [full(1, H), pl.BlockSpec((1, 1, BN), nblk3),
               full(H, LAT), full(1, LAT), full(H, LAT), full(1, LAT),
               full(LAT, H), full(1, H), full(B, LAT),
               pl.BlockSpec((H, _DW), dblk), pl.BlockSpec((1, _DW), dblk),
               full(H, MAXN * MAXN), full(1, MAXN * MAXN)]
        ),
        out_specs=[pl.BlockSpec((B, _DW), dblk),
                   full(B, MAXN * MAXN), full(B, LAT), full(B, LAT)],
        out_shape=[jax.ShapeDtypeStruct((B, MAXN * IN), jnp.float32),
                   jax.ShapeDtypeStruct((B, MAXN * MAXN), jnp.float32),
                   jax.ShapeDtypeStruct((B, LAT), jnp.float32),
                   jax.ShapeDtypeStruct((B, LAT), jnp.float32)],
        scratch_shapes=[pltpu.VMEM((B, H), jnp.float32),
                        pltpu.VMEM((B, FC), jnp.float32),
                        pltpu.VMEM((B, H), jnp.float32)],
    )(*aggs, dega3, degb3, b2r, batch3, Wmu, bmur, Wlv, blvr,
      Wd1, bd1r, eps, Wn, bnr, Wa, bar)


def kernel(x, edge_index, batch, W1, b1, W2, b2, Wmu, bmu, Wlv, blv,
           Wd1, bd1, Wn, bn, Wa, ba):
    npad = EPAD - E
    fill = jnp.arange(npad, dtype=jnp.int32)
    src_pad = jnp.concatenate([edge_index[0], fill % N])
    dst_pad = jnp.concatenate([edge_index[1], N + fill % (NR - N)])

    zeros_1d = jnp.zeros((NR,), jnp.float32)
    ones_w = jnp.ones((DKW,), jnp.float32)

    dega, degb = _deg_call(dst_pad.reshape(-1, DKW), ones_w, zeros_1d)
    dega3 = dega[:N].reshape(NB, 1, BN)
    degb3 = degb[:N].reshape(NB, 1, BN)

    src2 = src_pad.reshape(-1, KW)
    dst2 = dst_pad.reshape(-1, KW)
    ps1 = _mm1(x, W1, dega3, degb3)
    aggs1 = _agg_call(*ps1, src2, dst2)

    ps2 = _mm2(aggs1, dega3, degb3, b1.reshape(1, H), W2)
    aggs2 = _agg_call(*ps2, src2, dst2)

    eps = jnp.asarray(_EPS)
    batch3 = batch.reshape(NB, 1, BN)
    xr, adj, mu, logvar = _tail(
        aggs2, dega3, degb3, b2.reshape(1, H), batch3,
        Wmu, bmu.reshape(1, LAT), Wlv, blv.reshape(1, LAT),
        Wd1, bd1.reshape(1, H), eps, Wn, bn.reshape(1, MAXN * IN),
        Wa, ba.reshape(1, MAXN * MAXN))
    return (xr.reshape(B, MAXN, IN), adj.reshape(B, MAXN, MAXN), mu, logvar)


# speedup vs baseline: 1.0637x; 1.0001x over previous
import jax
import jax.numpy as jnp
import numpy as np
from jax import lax
from jax.experimental import pallas as pl
from jax.experimental.pallas import tpu as pltpu
from jax.experimental.pallas import tpu_sc as plsc

N = 10000
E = 160000
IN = 256
H = 512
LAT = 128
MAXN = 32
B = 64

NC = 2
NS = 16
KW = 64
NR = 10240
STRIPE = NR // NS
EPAD = 163840
FC = 128
NFC = H // FC
BN = 400
NB = N // BN

_mesh = plsc.VectorSubcoreMesh(core_axis_name="c", subcore_axis_name="s")

_EPS = np.asarray(
    jax.random.normal(jax.random.key(42), (B, LAT), jnp.float32))


DKW = 128
DWIN = EPAD // (NC * NS) // DKW


def _deg_body(dst128_hbm, ones_hbm, zeros_hbm, dega, degb,
              acc, ones_v, idx_v, sem_s):
    c = lax.axis_index("c")
    s = lax.axis_index("s")
    t = c * NS + s
    pltpu.sync_copy(ones_hbm, ones_v)
    pltpu.sync_copy(zeros_hbm.at[pl.ds(s * STRIPE, STRIPE)],
                    acc.at[pl.ds(s * STRIPE, STRIPE)])
    pltpu.sync_copy(dst128_hbm.at[pl.ds(t * DWIN, DWIN)], idx_v)
    plsc.subcore_barrier()

    def fire(i, carry):
        pltpu.async_copy(ones_v, acc.at[idx_v.at[i]], sem_s, add=True)
        return carry

    lax.fori_loop(0, DWIN, fire, 0)

    def drain(i, carry):
        pltpu.make_async_copy(zeros_hbm.at[pl.ds(0, DKW)], ones_v,
                              sem_s).wait()
        return carry

    lax.fori_loop(0, DWIN, drain, 0)
    plsc.subcore_barrier()

    @pl.when(c == 0)
    def _():
        pltpu.sync_copy(acc.at[pl.ds(s * STRIPE, STRIPE)],
                        dega.at[pl.ds(s * STRIPE, STRIPE)])

    @pl.when(c == 1)
    def _():
        pltpu.sync_copy(acc.at[pl.ds(s * STRIPE, STRIPE)],
                        degb.at[pl.ds(s * STRIPE, STRIPE)])


_deg_call = pl.kernel(
    _deg_body,
    out_type=[jax.ShapeDtypeStruct((NR,), jnp.float32),
              jax.ShapeDtypeStruct((NR,), jnp.float32)],
    mesh=_mesh,
    scratch_types=[
        pltpu.VMEM_SHARED((NR,), jnp.float32),
        pltpu.VMEM((DKW,), jnp.float32),
        pltpu.VMEM((DWIN, DKW), jnp.int32),
        pltpu.SemaphoreType.DMA,
    ],
)


NWIN = EPAD // NS // KW
IB = 16
NBUF = 5
GD = 4
SD = 1


def _agg_chunk(p_hbm, src2_hbm, dst2_hbm, out_hbm,
               acc, rows_v, sidx_v, didx_v, sem_g, sem_s, s):
    @pl.when(s < NS - 1)
    def _():
        pltpu.sync_copy(p_hbm.at[pl.ds(s * STRIPE, STRIPE)],
                        acc.at[pl.ds(s * STRIPE, STRIPE)])

    @pl.when(s == NS - 1)
    def _():
        pltpu.sync_copy(p_hbm.at[pl.ds((NS - 1) * STRIPE, N - (NS - 1) * STRIPE)],
                        acc.at[pl.ds((NS - 1) * STRIPE, N - (NS - 1) * STRIPE)])

    def load_idx(bi, buf):
        pltpu.sync_copy(src2_hbm.at[pl.ds(s * NWIN + bi * IB, IB)],
                        sidx_v.at[buf])
        pltpu.sync_copy(dst2_hbm.at[pl.ds(s * NWIN + bi * IB, IB)],
                        didx_v.at[buf])

    def start_gather(w, buf):
        pltpu.async_copy(p_hbm.at[sidx_v.at[(w // IB) % 2, w % IB]],
                         rows_v.at[buf], sem_g)

    def wait_gather(buf):
        pltpu.make_async_copy(p_hbm.at[pl.ds(0, KW)], rows_v.at[buf],
                              sem_g).wait()

    def start_scatter(w, buf):
        pltpu.async_copy(rows_v.at[buf],
                         acc.at[didx_v.at[(w // IB) % 2, w % IB]], sem_s,
                         add=True)

    def wait_scatter(buf):
        pltpu.make_async_copy(p_hbm.at[pl.ds(0, KW)], rows_v.at[buf],
                              sem_s).wait()

    load_idx(0, 0)
    plsc.subcore_barrier()
    for k in range(GD):
        start_gather(k, k)

    def body(g, carry):
        for j in range(NBUF):
            w = g * NBUF + j
            wait_gather(j)

            @pl.when(w >= SD)
            def _():
                wait_scatter((j + NBUF - SD) % NBUF)

            @pl.when((w + GD < NWIN) & ((w + GD) % IB == 0))
            def _():
                load_idx((w + GD) // IB, ((w + GD) // IB) % 2)

            @pl.when(w + GD < NWIN)
            def _():
                start_gather(w + GD, (j + GD) % NBUF)

            start_scatter(w, j)
        return carry

    lax.fori_loop(0, NWIN // NBUF, body, 0)
    for k in range(SD):
        wait_scatter(NBUF - SD + k)
    plsc.subcore_barrier()
    pltpu.sync_copy(acc.at[pl.ds(s * STRIPE, STRIPE)],
                    out_hbm.at[pl.ds(s * STRIPE, STRIPE)])
    plsc.subcore_barrier()


def _agg_body(p0, p1, p2, p3, src2_hbm, dst2_hbm,
              a0, a1, a2, a3,
              acc, rows_v, sidx_v, didx_v, sem_g, sem_s):
    c = lax.axis_index("c")
    s = lax.axis_index("s")

    @pl.when(c == 0)
    def _():
        _agg_chunk(p0, src2_hbm, dst2_hbm, a0, acc, rows_v,
                   sidx_v, didx_v, sem_g, sem_s, s)
        _agg_chunk(p1, src2_hbm, dst2_hbm, a1, acc, rows_v,
                   sidx_v, didx_v, sem_g, sem_s, s)

    @pl.when(c == 1)
    def _():
        _agg_chunk(p2, src2_hbm, dst2_hbm, a2, acc, rows_v,
                   sidx_v, didx_v, sem_g, sem_s, s)
        _agg_chunk(p3, src2_hbm, dst2_hbm, a3, acc, rows_v,
                   sidx_v, didx_v, sem_g, sem_s, s)


_agg_call = pl.kernel(
    _agg_body,
    out_type=[jax.ShapeDtypeStruct((NR, FC), jnp.float32)] * NFC,
    mesh=_mesh,
    scratch_types=[
        pltpu.VMEM_SHARED((NR, FC), jnp.float32),
        pltpu.VMEM((NBUF, KW, FC), jnp.float32),
        pltpu.VMEM((2, IB, KW), jnp.int32),
        pltpu.VMEM((2, IB, KW), jnp.int32),
        pltpu.SemaphoreType.DMA,
        pltpu.SemaphoreType.DMA,
    ],
)


def _mm1_body(x_ref, w_ref, dega_ref, degb_ref, o0, o1, o2, o3):
    dis = lax.rsqrt(dega_ref[0, 0, :] + degb_ref[0, 0, :] + 1.0)
    h = jnp.dot(x_ref[...], w_ref[...], preferred_element_type=jnp.float32)
    p = h * dis[:, None]
    o0[...] = p[:, 0 * FC:1 * FC]
    o1[...] = p[:, 1 * FC:2 * FC]
    o2[...] = p[:, 2 * FC:3 * FC]
    o3[...] = p[:, 3 * FC:4 * FC]


def _mm1(x, W1, dega3, degb3):
    return pl.pallas_call(
        _mm1_body,
        grid=(NB,),
        in_specs=[
            pl.BlockSpec((BN, IN), lambda i: (i, 0)),
            pl.BlockSpec((IN, H), lambda i: (0, 0)),
            pl.BlockSpec((1, 1, BN), lambda i: (i, 0, 0)),
            pl.BlockSpec((1, 1, BN), lambda i: (i, 0, 0)),
        ],
        out_specs=[pl.BlockSpec((BN, FC), lambda i: (i, 0))] * NFC,
        out_shape=[jax.ShapeDtypeStruct((N, FC), jnp.float32)] * NFC,
    )(x, W1, dega3, degb3)


def _mm2_body(a0, a1, a2, a3, dega_ref, degb_ref,
              b_ref, w_ref, o0, o1, o2, o3):
    dis = lax.rsqrt(dega_ref[0, 0, :] + degb_ref[0, 0, :] + 1.0)
    hcat = jnp.concatenate([a0[...], a1[...], a2[...], a3[...]], axis=1)
    h1 = jnp.maximum(hcat * dis[:, None] + b_ref[0, :], 0.0)
    h2 = jnp.dot(h1, w_ref[...], preferred_element_type=jnp.float32)
    p = h2 * dis[:, None]
    o0[...] = p[:, 0 * FC:1 * FC]
    o1[...] = p[:, 1 * FC:2 * FC]
    o2[...] = p[:, 2 * FC:3 * FC]
    o3[...] = p[:, 3 * FC:4 * FC]


def _mm2(aggs, dega3, degb3, b1r, W2):
    return pl.pallas_call(
        _mm2_body,
        grid=(NB,),
        in_specs=(
            [pl.BlockSpec((BN, FC), lambda i: (i, 0))] * NFC
            + [pl.BlockSpec((1, 1, BN), lambda i: (i, 0, 0))] * 2
            + [pl.BlockSpec((1, H), lambda i: (0, 0)),
               pl.BlockSpec((H, H), lambda i: (0, 0))]
        ),
        out_specs=[pl.BlockSpec((BN, FC), lambda i: (i, 0))] * NFC,
        out_shape=[jax.ShapeDtypeStruct((N, FC), jnp.float32)] * NFC,
    )(*aggs, dega3, degb3, b1r, W2)


_NXC = 8
_DW = MAXN * IN // _NXC


def _tail_body(a0, a1, a2, a3, dega_ref, degb_ref,
               b_ref, batch_ref, wmu_ref, bmu_ref, wlv_ref, blv_ref,
               wd1_ref, bd1_ref, eps_ref, wn_ref, bn_ref, wa_ref, ba_ref,
               xr_out, adj_out, mu_out, lv_out, s_scr, c_scr, hd_scr):
    i = pl.program_id(0)

    @pl.when(i == 0)
    def _():
        s_scr[...] = jnp.zeros_like(s_scr)
        c_scr[...] = jnp.zeros_like(c_scr)

    @pl.when(i < NB)
    def _():
        dis = lax.rsqrt(dega_ref[0, 0, :] + degb_ref[0, 0, :] + 1.0)
        hcat = jnp.concatenate([a0[...], a1[...], a2[...], a3[...]], axis=1)
        h2 = jnp.maximum(hcat * dis[:, None] + b_ref[0, :], 0.0)
        bt = batch_ref[0, 0, :]
        oh = (bt[None, :] == lax.broadcasted_iota(jnp.int32, (B, BN), 0)
              ).astype(jnp.float32)
        s_scr[...] += jnp.dot(oh, h2, preferred_element_type=jnp.float32)
        c_scr[...] += jnp.sum(oh, axis=1, keepdims=True)

    @pl.when(i == NB - 1)
    def _():
        g = s_scr[...] / jnp.maximum(c_scr[:, 0:1], 1.0)
        mu = jnp.dot(g, wmu_ref[...], preferred_element_type=jnp.float32) + bmu_ref[0, :]
        lv = jnp.dot(g, wlv_ref[...], preferred_element_type=jnp.float32) + blv_ref[0, :]
        std = jnp.exp(0.5 * lv)
        z = mu + eps_ref[...] * std
        hd_scr[...] = jnp.maximum(
            jnp.dot(z, wd1_ref[...], preferred_element_type=jnp.float32) + bd1_ref[0, :],
            0.0)
        mu_out[...] = mu
        lv_out[...] = lv

    @pl.when((i >= NB) & (i < NB + _NXC))
    def _():
        xr_out[...] = jnp.dot(hd_scr[...], wn_ref[...],
                              preferred_element_type=jnp.float32) + bn_ref[0, :]

    @pl.when(i == NB + _NXC)
    def _():
        a = jnp.dot(hd_scr[...], wa_ref[...],
                    preferred_element_type=jnp.float32) + ba_ref[0, :]
        sg = 1.0 / (1.0 + jnp.exp(-a))
        ci = lax.broadcasted_iota(jnp.int32, (B, MAXN * MAXN), 1)
        diag = (ci // MAXN) == (ci % MAXN)
        adj_out[...] = jnp.where(diag, 0.0, sg)


def _tail(aggs, dega3, degb3, b2r, batch3, Wmu, bmur, Wlv, blvr,
          Wd1, bd1r, eps, Wn, bnr, Wa, bar):
    full = lambda a, b: pl.BlockSpec((a, b), lambda i: (0, 0))
    nblk = lambda i: (jnp.minimum(i, NB - 1), 0)
    nblk3 = lambda i: (jnp.minimum(i, NB - 1), 0, 0)
    dblk = lambda i: (0, jnp.clip(i - NB, 0, _NXC - 1))
    return pl.pallas_call(
        _tail_body,
        grid=(NB + _NXC + 1,),
        in_specs=(
            [pl.BlockSpec((BN, FC), nblk)] * NFC
            + [pl.BlockSpec((1, 1, BN), nblk3)] * 2
            + [full(1, H), pl.BlockSpec((1, 1, BN), nblk3),
               full(H, LAT), full(1, LAT), full(H, LAT), full(1, LAT),
               full(LAT, H), full(1, H), full(B, LAT),
               pl.BlockSpec((H, _DW), dblk), pl.BlockSpec((1, _DW), dblk),
               full(H, MAXN * MAXN), full(1, MAXN * MAXN)]
        ),
        out_specs=[pl.BlockSpec((B, _DW), dblk),
                   full(B, MAXN * MAXN), full(B, LAT), full(B, LAT)],
        out_shape=[jax.ShapeDtypeStruct((B, MAXN * IN), jnp.float32),
                   jax.ShapeDtypeStruct((B, MAXN * MAXN), jnp.float32),
                   jax.ShapeDtypeStruct((B, LAT), jnp.float32),
                   jax.ShapeDtypeStruct((B, LAT), jnp.float32)],
        scratch_shapes=[pltpu.VMEM((B, H), jnp.float32),
                        pltpu.VMEM((B, FC), jnp.float32),
                        pltpu.VMEM((B, H), jnp.float32)],
    )(*aggs, dega3, degb3, b2r, batch3, Wmu, bmur, Wlv, blvr,
      Wd1, bd1r, eps, Wn, bnr, Wa, bar)


def kernel(x, edge_index, batch, W1, b1, W2, b2, Wmu, bmu, Wlv, blv,
           Wd1, bd1, Wn, bn, Wa, ba):
    npad = EPAD - E
    fill = jnp.arange(npad, dtype=jnp.int32)
    src_pad = jnp.concatenate([edge_index[0], fill % N])
    dst_pad = jnp.concatenate([edge_index[1], N + fill % (NR - N)])

    zeros_1d = jnp.zeros((NR,), jnp.float32)
    ones_w = jnp.ones((DKW,), jnp.float32)

    dega, degb = _deg_call(dst_pad.reshape(-1, DKW), ones_w, zeros_1d)
    dega3 = dega[:N].reshape(NB, 1, BN)
    degb3 = degb[:N].reshape(NB, 1, BN)

    src2 = src_pad.reshape(-1, KW)
    dst2 = dst_pad.reshape(-1, KW)
    ps1 = _mm1(x, W1, dega3, degb3)
    aggs1 = _agg_call(*ps1, src2, dst2)

    ps2 = _mm2(aggs1, dega3, degb3, b1.reshape(1, H), W2)
    aggs2 = _agg_call(*ps2, src2, dst2)

    eps = jnp.asarray(_EPS)
    batch3 = batch.reshape(NB, 1, BN)
    xr, adj, mu, logvar = _tail(
        aggs2, dega3, degb3, b2.reshape(1, H), batch3,
        Wmu, bmu.reshape(1, LAT), Wlv, blv.reshape(1, LAT),
        Wd1, bd1.reshape(1, H), eps, Wn, bn.reshape(1, MAXN * IN),
        Wa, ba.reshape(1, MAXN * MAXN))
    return (xr.reshape(B, MAXN, IN), adj.reshape(B, MAXN, MAXN), mu, logvar)
